# baseline (speedup 1.0000x reference)
import jax
import jax.numpy as jnp
from jax import lax
from jax.experimental import pallas as pl
from jax.experimental.pallas import tpu as pltpu
from jax.experimental.pallas import tpu_sc as plsc

_NC = 2
_NS = 16
_LANES = 16
_NW = _NC * _NS
_NSUB = 4


def _quantize_body(x_hbm, levels_hbm, q_hbm, idx_hbm,
                   x_v, q_v, idx_v, sems_in, sem_out):
    sub = x_v.shape[1]
    chunk = _NSUB * sub
    wid = lax.axis_index("s") * _NC + lax.axis_index("c")
    base = wid * chunk

    copies_in = [
        pltpu.async_copy(x_hbm.at[pl.ds(base + s * sub, sub)], x_v.at[s],
                         sems_in.at[s])
        for s in range(_NSUB)
    ]
    copies_out = []
    for s in range(_NSUB):
        copies_in[s].wait()
        x_s, q_s, idx_s = x_v.at[s], q_v.at[s], idx_v.at[s]

        @plsc.parallel_loop(0, sub, _LANES, unroll=8)
        def body(off):
            xv = x_s[pl.ds(off, _LANES)]
            u = 255.0 / (1.0 + jnp.exp(xv * -2.0))
            iv = (u + 0.5).astype(jnp.int32)
            q_s[pl.ds(off, _LANES)] = iv.astype(jnp.float32) * (1.0 / 127.5) - 1.0
            idx_s[pl.ds(off, _LANES)] = iv

        copies_out.append(pltpu.async_copy(
            q_s, q_hbm.at[pl.ds(base + s * sub, sub)], sem_out))
        copies_out.append(pltpu.async_copy(
            idx_s, idx_hbm.at[pl.ds(base + s * sub, sub)], sem_out))
    for c in copies_out:
        c.wait()


def kernel(x, levels):
    n = x.shape[0]
    chunk = n // _NW
    sub = chunk // _NSUB
    xf = x.reshape(n)
    q, idx = pl.kernel(
        _quantize_body,
        out_type=[
            jax.ShapeDtypeStruct((n,), jnp.float32),
            jax.ShapeDtypeStruct((n,), jnp.int32),
        ],
        mesh=plsc.VectorSubcoreMesh(
            core_axis_name="c", subcore_axis_name="s",
            num_cores=_NC, num_subcores=_NS,
        ),
        scratch_types=[
            pltpu.VMEM((_NSUB, sub), jnp.float32),
            pltpu.VMEM((_NSUB, sub), jnp.float32),
            pltpu.VMEM((_NSUB, sub), jnp.int32),
            pltpu.SemaphoreType.DMA((_NSUB,)),
            pltpu.SemaphoreType.DMA,
        ],
    )(xf, levels)
    return q.reshape(n, 1), idx.reshape(n, 1)


# speedup vs baseline: 1.0615x; 1.0615x over previous
import jax
import jax.numpy as jnp
from jax import lax
from jax.experimental import pallas as pl
from jax.experimental.pallas import tpu as pltpu
from jax.experimental.pallas import tpu_sc as plsc

_NC = 2
_NS = 16
_LANES = 16
_NW = _NC * _NS


def _quantize_body(x_hbm, levels_hbm, q_hbm, idx_hbm, x_v, q_v, idx_v):
    chunk = x_v.shape[0]
    wid = lax.axis_index("s") * _NC + lax.axis_index("c")
    base = wid * chunk
    pltpu.sync_copy(x_hbm.at[pl.ds(base, chunk)], x_v)

    @plsc.parallel_loop(0, chunk, _LANES, unroll=8)
    def body(off):
        xv = x_v[pl.ds(off, _LANES)]
        u = 255.0 / (1.0 + jnp.exp(xv * -2.0))
        iv = (u + 0.5).astype(jnp.int32)
        q_v[pl.ds(off, _LANES)] = iv.astype(jnp.float32) * (1.0 / 127.5) - 1.0
        idx_v[pl.ds(off, _LANES)] = iv

    pltpu.sync_copy(q_v, q_hbm.at[pl.ds(base, chunk)])
    pltpu.sync_copy(idx_v, idx_hbm.at[pl.ds(base, chunk)])


def kernel(x, levels):
    n = x.shape[0]
    chunk = n // _NW
    xf = x.reshape(n)
    q, idx = pl.kernel(
        _quantize_body,
        out_type=[
            jax.ShapeDtypeStruct((n,), jnp.float32),
            jax.ShapeDtypeStruct((n,), jnp.int32),
        ],
        mesh=plsc.VectorSubcoreMesh(
            core_axis_name="c", subcore_axis_name="s",
            num_cores=_NC, num_subcores=_NS,
        ),
        scratch_types=[
            pltpu.VMEM((chunk,), jnp.float32),
            pltpu.VMEM((chunk,), jnp.float32),
            pltpu.VMEM((chunk,), jnp.int32),
        ],
    )(xf, levels)
    return q.reshape(n, 1), idx.reshape(n, 1)
